# baseline (speedup 1.0000x reference)
import functools

import jax
import jax.numpy as jnp
from jax import lax
from jax.experimental import pallas as pl
from jax.experimental.pallas import tpu as pltpu
from jax.experimental.pallas import tpu_sc as plsc

NC, NS, L = 2, 16, 16
NW = NC * NS
B = 16384
D = 32
K = 7
NPAIR = K - 1
RPT = B // NW
CHUNK = 128
NCHUNK = RPT // CHUNK

@functools.cache
def _make_sc_dots():
    mesh = plsc.VectorSubcoreMesh(
        core_axis_name="c", subcore_axis_name="s", num_cores=NC, num_subcores=NS
    )
    return pl.kernel(
        _sc_dots_body,
        out_type=jax.ShapeDtypeStruct((NW, NPAIR * RPT), jnp.float32),
        mesh=mesh,
        compiler_params=pltpu.CompilerParams(
            use_tc_tiling_on_sc=False, needs_layout_passes=False
        ),
        scratch_types=[
            pltpu.VMEM((K, NCHUNK, CHUNK), jnp.int32),
            pltpu.VMEM((CHUNK, D), jnp.float32),
            pltpu.VMEM((NPAIR, CHUNK, D), jnp.float32),
            pltpu.VMEM((NPAIR * RPT,), jnp.float32),
            pltpu.SemaphoreType.DMA,
        ],
    )


def _sc_dots_body(batch_hbm, node_hbm, ctx_hbm, out_hbm, idx_v, vi_v, ctx_v, dots_v, sem):
    wid = lax.axis_index("s") * NC + lax.axis_index("c")
    pltpu.sync_copy(batch_hbm.at[wid], idx_v)

    def chunk_body(i, carry):
        cps = [pltpu.async_copy(node_hbm.at[idx_v.at[0, i]], vi_v, sem)]
        for c in range(NPAIR):
            cps.append(
                pltpu.async_copy(ctx_hbm.at[idx_v.at[c + 1, i]], ctx_v.at[c], sem)
            )
        for cp in cps:
            cp.wait()

        iota = lax.iota(jnp.int32, L)

        def group_body(g, gcarry):
            rows = g * L + iota
            acc = [jnp.zeros((L,), jnp.float32) for _ in range(NPAIR)]
            for d in range(D):
                cold = jnp.full((L,), d, jnp.int32)
                vi_d = plsc.load_gather(vi_v, [rows, cold])
                for c in range(NPAIR):
                    ctx_d = plsc.load_gather(
                        ctx_v, [jnp.full((L,), c, jnp.int32), rows, cold]
                    )
                    acc[c] = acc[c] + vi_d * ctx_d
            base = i * CHUNK + g * L
            dots_v[pl.ds(base, L)] = acc[0]
            for c in range(1, NPAIR):
                dots_v[pl.ds(c * RPT + base, L)] = -acc[c]
            return gcarry

        return lax.fori_loop(0, CHUNK // L, group_body, carry)

    lax.fori_loop(0, NCHUNK, chunk_body, 0)
    pltpu.sync_copy(dots_v, out_hbm.at[wid])


def _tc_loss_body(x_ref, o_ref):
    x = x_ref[...]
    ls = jnp.minimum(x, 0.0) - jnp.log1p(jnp.exp(-jnp.abs(x)))
    o_ref[0, 0] = -jnp.sum(ls)


def kernel(batch, node_embed, context_node_embed):
    idx = batch.astype(jnp.int32)
    idx_t = idx.T.reshape(K, NW, NCHUNK, CHUNK).transpose(1, 0, 2, 3)
    dots = _make_sc_dots()(idx_t, node_embed, context_node_embed)
    x = dots.reshape(B * NPAIR // 128, 128)
    loss = pl.pallas_call(
        _tc_loss_body,
        out_shape=jax.ShapeDtypeStruct((1, 1), jnp.float32),
        out_specs=pl.BlockSpec(memory_space=pltpu.SMEM),
    )(x)
    return loss[0, 0]
